# baseline (speedup 1.0000x reference)
import jax
import jax.numpy as jnp
from jax import lax
from jax.experimental import pallas as pl
from jax.experimental.pallas import tpu as pltpu
from jax.experimental.pallas import tpu_sc as plsc

K_NN = 16
SAMPLING_RATIO = 0.25
EPS = 1e-5
_NW = 32
_INTERPRET = False


def _bn_relu(h, gamma, beta):
    mean = jnp.mean(h, axis=(0, 1))
    var = jnp.var(h, axis=(0, 1))
    hn = (h - mean) / jnp.sqrt(var + EPS)
    return jax.nn.relu(hn * gamma + beta)


def _mm_stats_body(x_ref, w_ref, z_ref, st_ref):
    b = pl.program_id(0)
    z = lax.dot_general(x_ref[0], w_ref[...], (((1,), (1,)), ((), ())),
                        preferred_element_type=jnp.float32)
    z_ref[0] = z
    s0 = jnp.sum(z, axis=0, keepdims=True)
    s1 = jnp.sum(z * z, axis=0, keepdims=True)
    upd = jnp.concatenate(
        [s0, s1, jnp.zeros((6, z.shape[1]), jnp.float32)], axis=0)

    @pl.when(b == 0)
    def _():
        st_ref[...] = jnp.zeros_like(st_ref)

    st_ref[...] += upd


def _bn_mm_stats_body(z_ref, stin_ref, g_ref, b_ref, w_ref, z2_ref, st_ref,
                      *, n_tot):
    b = pl.program_id(0)
    mean = stin_ref[0:1, :] * (1.0 / n_tot)
    var = stin_ref[1:2, :] * (1.0 / n_tot) - mean * mean
    h = (z_ref[0] - mean) / jnp.sqrt(var + EPS) * g_ref[...] + b_ref[...]
    h = jnp.maximum(h, 0.0)
    z2 = lax.dot_general(h, w_ref[...], (((1,), (1,)), ((), ())),
                         preferred_element_type=jnp.float32)
    z2_ref[0] = z2
    s0 = jnp.sum(z2, axis=0, keepdims=True)
    s1 = jnp.sum(z2 * z2, axis=0, keepdims=True)
    upd = jnp.concatenate(
        [s0, s1, jnp.zeros((6, z2.shape[1]), jnp.float32)], axis=0)

    @pl.when(b == 0)
    def _():
        st_ref[...] = jnp.zeros_like(st_ref)

    st_ref[...] += upd


def _bn_body(z_ref, stin_ref, g_ref, b_ref, h_ref, *, n_tot):
    mean = stin_ref[0:1, :] * (1.0 / n_tot)
    var = stin_ref[1:2, :] * (1.0 / n_tot) - mean * mean
    h = (z_ref[0] - mean) / jnp.sqrt(var + EPS) * g_ref[...] + b_ref[...]
    h_ref[0] = jnp.maximum(h, 0.0)


def _knn_body(d2_ref, nb_ref):
    d = d2_ref[0]
    iota = lax.broadcasted_iota(jnp.int32, d.shape, 1).astype(jnp.float32)
    cols = []
    for _ in range(K_NN):
        rowmin = jnp.min(d, axis=1, keepdims=True)
        am = jnp.min(jnp.where(d == rowmin, iota, jnp.float32(4096.0)),
                     axis=1, keepdims=True)
        cols.append(am)
        d = jnp.where(iota == am, jnp.float32(3e38), d)
    nb_ref[0] = jnp.concatenate(cols, axis=1).astype(jnp.int32)


def _sc_pool_body(h_hbm, idx_hbm, y_hbm, idx_v, rows_a, rows_b, out_v,
                  sem_a, sem_b, *, q_per_w, n_chan):
    c = lax.axis_index("c")
    s = lax.axis_index("s")
    wid = s * 2 + c
    base = wid * q_per_w
    pltpu.sync_copy(idx_hbm.at[pl.ds(base, q_per_w)], idx_v)
    pltpu.async_copy(h_hbm.at[idx_v.at[0]], rows_a, sem_a)

    nch = n_chan // 16

    def _pool(rows_ref, q):
        for cc in range(nch):
            m = rows_ref[0, pl.ds(cc * 16, 16)]
            for k in range(1, K_NN):
                m = jnp.maximum(m, rows_ref[k, pl.ds(cc * 16, 16)])
            out_v[q, pl.ds(cc * 16, 16)] = m

    def _body(t, carry):
        qa = 2 * t
        qb = qa + 1
        pltpu.async_copy(h_hbm.at[idx_v.at[qb]], rows_b, sem_b)
        pltpu.make_async_copy(h_hbm.at[idx_v.at[qa]], rows_a, sem_a).wait()
        _pool(rows_a, qa)

        @pl.when(t < q_per_w // 2 - 1)
        def _():
            pltpu.async_copy(h_hbm.at[idx_v.at[qa + 2]], rows_a, sem_a)

        pltpu.make_async_copy(h_hbm.at[idx_v.at[qb]], rows_b, sem_b).wait()
        _pool(rows_b, qb)
        return carry

    lax.fori_loop(0, q_per_w // 2, _body, 0)
    pltpu.sync_copy(out_v, y_hbm.at[pl.ds(base, q_per_w)])


def kernel(x, p, W1, g1, b1, W2, g2, b2, Ws, bs):
    B, N, Cin = x.shape
    C = W1.shape[0]
    M = int(N * SAMPLING_RATIO)
    n_tot = float(B * N)

    hx = _bn_relu(jnp.einsum('bnc,oc->bno', x, W1), g1, b1)
    hx = _bn_relu(jnp.einsum('bnc,oc->bno', hx, W2), g2, b2)
    scores = jnp.einsum('bnc,oc->bno', hx, Ws) + bs
    _, topk_idx = lax.top_k(scores[..., 0], M)
    p_out = jnp.take_along_axis(
        p, topk_idx.reshape(B, -1)[:, :, None], axis=1).reshape(B, M, 3)
    d2 = (jnp.sum(p_out ** 2, axis=-1)[:, :, None]
          + jnp.sum(p ** 2, axis=-1)[:, None, :]
          - 2.0 * jnp.einsum('bmd,bnd->bmn', p_out, p))

    import functools
    z1, st1 = pl.pallas_call(
        _mm_stats_body,
        grid=(B,),
        in_specs=[pl.BlockSpec((1, N, Cin), lambda b: (b, 0, 0)),
                  pl.BlockSpec((C, Cin), lambda b: (0, 0))],
        out_specs=[pl.BlockSpec((1, N, C), lambda b: (b, 0, 0)),
                   pl.BlockSpec((8, C), lambda b: (0, 0))],
        out_shape=[jax.ShapeDtypeStruct((B, N, C), jnp.float32),
                   jax.ShapeDtypeStruct((8, C), jnp.float32)],
        interpret=_INTERPRET,
    )(x, W1)

    z2, st2 = pl.pallas_call(
        functools.partial(_bn_mm_stats_body, n_tot=n_tot),
        grid=(B,),
        in_specs=[pl.BlockSpec((1, N, C), lambda b: (b, 0, 0)),
                  pl.BlockSpec((8, C), lambda b: (0, 0)),
                  pl.BlockSpec((1, C), lambda b: (0, 0)),
                  pl.BlockSpec((1, C), lambda b: (0, 0)),
                  pl.BlockSpec((C, C), lambda b: (0, 0))],
        out_specs=[pl.BlockSpec((1, N, C), lambda b: (b, 0, 0)),
                   pl.BlockSpec((8, C), lambda b: (0, 0))],
        out_shape=[jax.ShapeDtypeStruct((B, N, C), jnp.float32),
                   jax.ShapeDtypeStruct((8, C), jnp.float32)],
        interpret=_INTERPRET,
    )(z1, st1, g1.reshape(1, C), b1.reshape(1, C), W2)

    h2p = pl.pallas_call(
        functools.partial(_bn_body, n_tot=n_tot),
        grid=(B,),
        in_specs=[pl.BlockSpec((1, N, C), lambda b: (b, 0, 0)),
                  pl.BlockSpec((8, C), lambda b: (0, 0)),
                  pl.BlockSpec((1, C), lambda b: (0, 0)),
                  pl.BlockSpec((1, C), lambda b: (0, 0))],
        out_specs=pl.BlockSpec((1, N, C), lambda b: (b, 0, 0)),
        out_shape=jax.ShapeDtypeStruct((B, N, C), jnp.float32),
        interpret=_INTERPRET,
    )(z2, st2, g2.reshape(1, C), b2.reshape(1, C))

    nb = pl.pallas_call(
        _knn_body,
        grid=(B,),
        in_specs=[pl.BlockSpec((1, M, N), lambda b: (b, 0, 0))],
        out_specs=pl.BlockSpec((1, M, K_NN), lambda b: (b, 0, 0)),
        out_shape=jax.ShapeDtypeStruct((B, M, K_NN), jnp.int32),
        interpret=_INTERPRET,
    )(d2)

    nb_flat = (nb + (jnp.arange(B, dtype=jnp.int32) * N)[:, None, None]
               ).reshape(B * M, K_NN)
    h_flat = h2p.reshape(B * N, C)

    if _INTERPRET:
        feats = jnp.take_along_axis(
            h_flat[None], nb_flat.reshape(1, -1)[:, :, None], axis=1)
        y_flat = jnp.max(feats.reshape(B * M, K_NN, C), axis=1)
    else:
        import functools as _ft
        q_per_w = (B * M) // _NW
        mesh = plsc.VectorSubcoreMesh(core_axis_name="c", subcore_axis_name="s")
        y_flat = pl.kernel(
            _ft.partial(_sc_pool_body, q_per_w=q_per_w, n_chan=C),
            out_type=jax.ShapeDtypeStruct((B * M, C), jnp.float32),
            mesh=mesh,
            scratch_types=[
                pltpu.VMEM((q_per_w, K_NN), jnp.int32),
                pltpu.VMEM((K_NN, C), jnp.float32),
                pltpu.VMEM((K_NN, C), jnp.float32),
                pltpu.VMEM((q_per_w, C), jnp.float32),
                pltpu.SemaphoreType.DMA,
                pltpu.SemaphoreType.DMA,
            ],
        )(h_flat, nb_flat)

    y = y_flat.reshape(B, M, C)
    return (y, p_out)
